# baseline (speedup 1.0000x reference)
import functools

import jax
import jax.numpy as jnp
from jax import lax
from jax.experimental import pallas as pl
from jax.experimental.pallas import tpu as pltpu
from jax.experimental.pallas import tpu_sc as plsc

N = 10000
F = 256
FH = 128
E = 160000
NC = 2
NS = 16
CHUNK = 128
CH = 80
E_TILE = CH * CHUNK
E_PAD = NS * E_TILE
NPAD = 10112
ZROWS = NPAD // NS
OROWS = 624
TAIL = N - NS * OROWS

NBUF = 2
PH = CH // 2


def _sc_body(x2, srcg, dstg, zrow, zcnt, agg_a, agg_b, cnt_a, cnt_b,
             idx_v, rows_v, ones_v, stage_v, agg_s, cnt_s,
             gsem0, gsem1, ssem0, ssem1):
    gsems = (gsem0, gsem1)
    ssems = (ssem0, ssem1)
    c = lax.axis_index("c")
    s = lax.axis_index("s")

    def load_idx(p):
        pltpu.sync_copy(srcg.at[pl.ds((c * NS + s) * CH + p * PH, PH)],
                        idx_v.at[pl.ds(0, PH)])
        pltpu.sync_copy(dstg.at[pl.ds(s * CH + p * PH, PH)],
                        idx_v.at[pl.ds(PH, PH)])

    def gather(l, b):
        pltpu.async_copy(x2.at[idx_v.at[l]], rows_v.at[b], gsems[b])

    def pair(t, lookahead):
        descs = []
        for b in range(2):
            l = 2 * t + b
            pltpu.make_async_copy(x2.at[idx_v.at[l]], rows_v.at[b],
                                  gsems[b]).wait()
            descs.append(pltpu.async_copy(
                rows_v.at[b], agg_s.at[idx_v.at[PH + l]], ssems[b],
                add=True))
        for b in range(2):
            @pl.when(c == b)
            def _():
                pltpu.sync_copy(ones_v, cnt_s.at[idx_v.at[PH + 2 * t + b]],
                                add=True)
        for b in range(2):
            descs[b].wait()
            if lookahead:
                gather(2 * t + b + 2, b)

    load_idx(0)
    gather(0, 0)
    gather(1, 1)
    pltpu.sync_copy(zrow, agg_s.at[pl.ds(s * ZROWS, ZROWS)])
    pltpu.sync_copy(zcnt, stage_v.at[pl.ds(0, ZROWS)])
    pltpu.sync_copy(stage_v.at[pl.ds(0, ZROWS)],
                    cnt_s.at[pl.ds(s * ZROWS, ZROWS)])
    for i in range(CHUNK // 16):
        ones_v[pl.ds(i * 16, 16)] = jnp.ones((16,), jnp.float32)
    plsc.subcore_barrier()

    for p in range(2):
        if p:
            load_idx(p)
            gather(0, 0)
            gather(1, 1)

        @pl.loop(0, PH // 2 - 1)
        def _ring(t):
            pair(t, lookahead=True)

        pair(PH // 2 - 1, lookahead=False)

    plsc.subcore_barrier()

    @pl.when(c == 0)
    def _():
        pltpu.sync_copy(agg_s.at[pl.ds(s * OROWS, OROWS)],
                        agg_a.at[pl.ds(s * OROWS, OROWS)])

        @pl.when(s == NS - 1)
        def _():
            pltpu.sync_copy(agg_s.at[pl.ds(NS * OROWS, TAIL)],
                            agg_a.at[pl.ds(NS * OROWS, TAIL)])

        @pl.when(s < 10)
        def _():
            pltpu.sync_copy(cnt_s.at[pl.ds(s * 1000, 1000)], stage_v)
            pltpu.sync_copy(stage_v, cnt_a.at[pl.ds(s * 1000, 1000)])

    @pl.when(c == 1)
    def _():
        pltpu.sync_copy(agg_s.at[pl.ds(s * OROWS, OROWS)],
                        agg_b.at[pl.ds(s * OROWS, OROWS)])

        @pl.when(s == NS - 1)
        def _():
            pltpu.sync_copy(agg_s.at[pl.ds(NS * OROWS, TAIL)],
                            agg_b.at[pl.ds(NS * OROWS, TAIL)])

        @pl.when(s < 10)
        def _():
            pltpu.sync_copy(cnt_s.at[pl.ds(s * 1000, 1000)], stage_v)
            pltpu.sync_copy(stage_v, cnt_b.at[pl.ds(s * 1000, 1000)])


@functools.lru_cache(maxsize=1)
def _sc_aggregate():
    mesh = plsc.VectorSubcoreMesh(core_axis_name="c", subcore_axis_name="s")
    return pl.kernel(
        _sc_body,
        out_type=(
            jax.ShapeDtypeStruct((N, FH), jnp.float32),
            jax.ShapeDtypeStruct((N, FH), jnp.float32),
            jax.ShapeDtypeStruct((N,), jnp.float32),
            jax.ShapeDtypeStruct((N,), jnp.float32),
        ),
        mesh=mesh,
        scratch_types=(
            pltpu.VMEM((2 * PH, CHUNK), jnp.int32),
            pltpu.VMEM((NBUF, CHUNK, FH), jnp.float32),
            pltpu.VMEM((CHUNK,), jnp.float32),
            pltpu.VMEM((1000,), jnp.float32),
            pltpu.VMEM_SHARED((NPAD, FH), jnp.float32),
            pltpu.VMEM_SHARED((NPAD,), jnp.float32),
            pltpu.SemaphoreType.DMA,
            pltpu.SemaphoreType.DMA,
            pltpu.SemaphoreType.DMA,
            pltpu.SemaphoreType.DMA,
        ),
    )


RB = 1000


def _tc_block(x_ref, aa_ref, ab_ref, ca_ref, cb_ref, wlt_ref, wrt_ref,
              w1t_ref, w2t_ref, bl_ref, b1_ref, b2_ref, g1_ref, bb1_ref,
              g2_ref, bb2_ref, out_ref):
    r = 1.0 / jnp.maximum(ca_ref[...] + cb_ref[...], 1.0)
    x = x_ref[...]
    y = jnp.dot(aa_ref[...] * r, wlt_ref[0:FH, :],
                preferred_element_type=jnp.float32)
    y = y + jnp.dot(ab_ref[...] * r, wlt_ref[FH:F, :],
                    preferred_element_type=jnp.float32)
    y = y + jnp.dot(x, wrt_ref[...], preferred_element_type=jnp.float32)
    z = y + bl_ref[...] + x
    mu = jnp.mean(z, axis=1, keepdims=True)
    zc = z - mu
    var = jnp.mean(zc * zc, axis=1, keepdims=True)
    y1 = zc * lax.rsqrt(var + 1e-5) * g1_ref[...] + bb1_ref[...]
    h = jnp.dot(y1, w1t_ref[...], preferred_element_type=jnp.float32)
    h = h + b1_ref[...]
    h = jnp.where(h > 0.0, h, jnp.exp(jnp.minimum(h, 0.0)) - 1.0)
    o = jnp.dot(h, w2t_ref[...], preferred_element_type=jnp.float32)
    o = o + b2_ref[...] + y1
    mu2 = jnp.mean(o, axis=1, keepdims=True)
    oc = o - mu2
    var2 = jnp.mean(oc * oc, axis=1, keepdims=True)
    out_ref[...] = oc * lax.rsqrt(var2 + 1e-5) * g2_ref[...] + bb2_ref[...]


def _row_spec(w):
    return pl.BlockSpec((RB, w), lambda i: (i, 0))


def _full_spec(h, w):
    return pl.BlockSpec((h, w), lambda i: (0, 0))


_tc_call = pl.pallas_call(
    _tc_block,
    grid=(N // RB,),
    in_specs=[
        _row_spec(F), _row_spec(FH), _row_spec(FH), _row_spec(1),
        _row_spec(1),
        _full_spec(F, F), _full_spec(F, F), _full_spec(F, F),
        _full_spec(F, F),
        _full_spec(1, F), _full_spec(1, F), _full_spec(1, F),
        _full_spec(1, F), _full_spec(1, F), _full_spec(1, F),
        _full_spec(1, F),
    ],
    out_specs=_row_spec(F),
    out_shape=jax.ShapeDtypeStruct((N, F), jnp.float32),
)


def kernel(x, edge_index, W_l, b_l, W_r, ln1_g, ln1_b, W1, b1, W2, b2,
           ln2_g, ln2_b):
    src = edge_index[0].astype(jnp.int32)
    dst = edge_index[1].astype(jnp.int32)
    pad = E_PAD - E
    src_p = jnp.concatenate([2 * src, jnp.zeros((pad,), jnp.int32)])
    dst_p = jnp.concatenate([dst, jnp.full((pad,), N, jnp.int32)])
    srcg = jnp.stack([src_p, src_p + 1]).reshape(NC * NS * CH, CHUNK)
    dstg = dst_p.reshape(NS * CH, CHUNK)
    x2 = x.reshape(2 * N, FH)
    zrow = jnp.zeros((ZROWS, FH), jnp.float32)
    zcnt = jnp.zeros((ZROWS,), jnp.float32)

    agg_a, agg_b, cnt_a, cnt_b = _sc_aggregate()(x2, srcg, dstg, zrow, zcnt)

    return _tc_call(
        x, agg_a, agg_b, cnt_a.reshape(N, 1), cnt_b.reshape(N, 1),
        W_l.T, W_r.T, W1.T, W2.T,
        b_l.reshape(1, F), b1.reshape(1, F), b2.reshape(1, F),
        ln1_g.reshape(1, F), ln1_b.reshape(1, F),
        ln2_g.reshape(1, F), ln2_b.reshape(1, F),
    )


# speedup vs baseline: 1.0727x; 1.0727x over previous
import functools

import jax
import jax.numpy as jnp
from jax import lax
from jax.experimental import pallas as pl
from jax.experimental.pallas import tpu as pltpu
from jax.experimental.pallas import tpu_sc as plsc

N = 10000
F = 256
FH = 128
E = 160000
NC = 2
NS = 16
CHUNK = 128
CH = 80
E_TILE = CH * CHUNK
E_PAD = NS * E_TILE
NPAD = 10112
ZROWS = NPAD // NS
OROWS = 624
TAIL = N - NS * OROWS

NBUF = 2
PH = CH // 2


def _sc_body(x2, srcg, dstg, zrow, zcnt, agg_a, agg_b, cnt_a, cnt_b,
             idx_v, rows_v, ones_v, stage_v, agg_s, cnt_s,
             gsem0, gsem1, ssem0, ssem1):
    gsems = (gsem0, gsem1)
    ssems = (ssem0, ssem1)
    c = lax.axis_index("c")
    s = lax.axis_index("s")

    def load_idx(p):
        pltpu.sync_copy(srcg.at[pl.ds((c * NS + s) * CH + p * PH, PH)],
                        idx_v.at[pl.ds(0, PH)])
        pltpu.sync_copy(dstg.at[pl.ds(s * CH + p * PH, PH)],
                        idx_v.at[pl.ds(PH, PH)])

    def gather(l, b):
        pltpu.async_copy(x2.at[idx_v.at[l]], rows_v.at[b], gsems[b])

    def pair(t, lookahead):
        for b in range(2):
            l = 2 * t + b
            pltpu.make_async_copy(x2.at[idx_v.at[l]], rows_v.at[b],
                                  gsems[b]).wait()
            pltpu.sync_copy(rows_v.at[b], agg_s.at[idx_v.at[PH + l]],
                            add=True)
            @pl.when(c == b)
            def _():
                pltpu.sync_copy(ones_v, cnt_s.at[idx_v.at[PH + l]],
                                add=True)
            if lookahead:
                gather(2 * t + b + 2, b)

    load_idx(0)
    gather(0, 0)
    gather(1, 1)
    pltpu.sync_copy(zrow, agg_s.at[pl.ds(s * ZROWS, ZROWS)])
    pltpu.sync_copy(zcnt, stage_v.at[pl.ds(0, ZROWS)])
    pltpu.sync_copy(stage_v.at[pl.ds(0, ZROWS)],
                    cnt_s.at[pl.ds(s * ZROWS, ZROWS)])
    for i in range(CHUNK // 16):
        ones_v[pl.ds(i * 16, 16)] = jnp.ones((16,), jnp.float32)
    plsc.subcore_barrier()

    for p in range(2):
        if p:
            load_idx(p)
            gather(0, 0)
            gather(1, 1)

        @pl.loop(0, PH // 2 - 1)
        def _ring(t):
            pair(t, lookahead=True)

        pair(PH // 2 - 1, lookahead=False)

    plsc.subcore_barrier()

    @pl.when(c == 0)
    def _():
        pltpu.sync_copy(agg_s.at[pl.ds(s * OROWS, OROWS)],
                        agg_a.at[pl.ds(s * OROWS, OROWS)])

        @pl.when(s == NS - 1)
        def _():
            pltpu.sync_copy(agg_s.at[pl.ds(NS * OROWS, TAIL)],
                            agg_a.at[pl.ds(NS * OROWS, TAIL)])

        @pl.when(s < 10)
        def _():
            pltpu.sync_copy(cnt_s.at[pl.ds(s * 1000, 1000)], stage_v)
            pltpu.sync_copy(stage_v, cnt_a.at[pl.ds(s * 1000, 1000)])

    @pl.when(c == 1)
    def _():
        pltpu.sync_copy(agg_s.at[pl.ds(s * OROWS, OROWS)],
                        agg_b.at[pl.ds(s * OROWS, OROWS)])

        @pl.when(s == NS - 1)
        def _():
            pltpu.sync_copy(agg_s.at[pl.ds(NS * OROWS, TAIL)],
                            agg_b.at[pl.ds(NS * OROWS, TAIL)])

        @pl.when(s < 10)
        def _():
            pltpu.sync_copy(cnt_s.at[pl.ds(s * 1000, 1000)], stage_v)
            pltpu.sync_copy(stage_v, cnt_b.at[pl.ds(s * 1000, 1000)])


@functools.lru_cache(maxsize=1)
def _sc_aggregate():
    mesh = plsc.VectorSubcoreMesh(core_axis_name="c", subcore_axis_name="s")
    return pl.kernel(
        _sc_body,
        out_type=(
            jax.ShapeDtypeStruct((N, FH), jnp.float32),
            jax.ShapeDtypeStruct((N, FH), jnp.float32),
            jax.ShapeDtypeStruct((N,), jnp.float32),
            jax.ShapeDtypeStruct((N,), jnp.float32),
        ),
        mesh=mesh,
        scratch_types=(
            pltpu.VMEM((2 * PH, CHUNK), jnp.int32),
            pltpu.VMEM((NBUF, CHUNK, FH), jnp.float32),
            pltpu.VMEM((CHUNK,), jnp.float32),
            pltpu.VMEM((1000,), jnp.float32),
            pltpu.VMEM_SHARED((NPAD, FH), jnp.float32),
            pltpu.VMEM_SHARED((NPAD,), jnp.float32),
            pltpu.SemaphoreType.DMA,
            pltpu.SemaphoreType.DMA,
            pltpu.SemaphoreType.DMA,
            pltpu.SemaphoreType.DMA,
        ),
    )


RB = 1000


def _tc_block(x_ref, aa_ref, ab_ref, ca_ref, cb_ref, wlt_ref, wrt_ref,
              w1t_ref, w2t_ref, bl_ref, b1_ref, b2_ref, g1_ref, bb1_ref,
              g2_ref, bb2_ref, out_ref):
    r = 1.0 / jnp.maximum(ca_ref[...] + cb_ref[...], 1.0)
    x = x_ref[...]
    y = jnp.dot(aa_ref[...] * r, wlt_ref[0:FH, :],
                preferred_element_type=jnp.float32)
    y = y + jnp.dot(ab_ref[...] * r, wlt_ref[FH:F, :],
                    preferred_element_type=jnp.float32)
    y = y + jnp.dot(x, wrt_ref[...], preferred_element_type=jnp.float32)
    z = y + bl_ref[...] + x
    mu = jnp.mean(z, axis=1, keepdims=True)
    zc = z - mu
    var = jnp.mean(zc * zc, axis=1, keepdims=True)
    y1 = zc * lax.rsqrt(var + 1e-5) * g1_ref[...] + bb1_ref[...]
    h = jnp.dot(y1, w1t_ref[...], preferred_element_type=jnp.float32)
    h = h + b1_ref[...]
    h = jnp.where(h > 0.0, h, jnp.exp(jnp.minimum(h, 0.0)) - 1.0)
    o = jnp.dot(h, w2t_ref[...], preferred_element_type=jnp.float32)
    o = o + b2_ref[...] + y1
    mu2 = jnp.mean(o, axis=1, keepdims=True)
    oc = o - mu2
    var2 = jnp.mean(oc * oc, axis=1, keepdims=True)
    out_ref[...] = oc * lax.rsqrt(var2 + 1e-5) * g2_ref[...] + bb2_ref[...]


def _row_spec(w):
    return pl.BlockSpec((RB, w), lambda i: (i, 0))


def _full_spec(h, w):
    return pl.BlockSpec((h, w), lambda i: (0, 0))


_tc_call = pl.pallas_call(
    _tc_block,
    grid=(N // RB,),
    in_specs=[
        _row_spec(F), _row_spec(FH), _row_spec(FH), _row_spec(1),
        _row_spec(1),
        _full_spec(F, F), _full_spec(F, F), _full_spec(F, F),
        _full_spec(F, F),
        _full_spec(1, F), _full_spec(1, F), _full_spec(1, F),
        _full_spec(1, F), _full_spec(1, F), _full_spec(1, F),
        _full_spec(1, F),
    ],
    out_specs=_row_spec(F),
    out_shape=jax.ShapeDtypeStruct((N, F), jnp.float32),
)


def kernel(x, edge_index, W_l, b_l, W_r, ln1_g, ln1_b, W1, b1, W2, b2,
           ln2_g, ln2_b):
    src = edge_index[0].astype(jnp.int32)
    dst = edge_index[1].astype(jnp.int32)
    pad = E_PAD - E
    src_p = jnp.concatenate([2 * src, jnp.zeros((pad,), jnp.int32)])
    dst_p = jnp.concatenate([dst, jnp.full((pad,), N, jnp.int32)])
    srcg = jnp.stack([src_p, src_p + 1]).reshape(NC * NS * CH, CHUNK)
    dstg = dst_p.reshape(NS * CH, CHUNK)
    x2 = x.reshape(2 * N, FH)
    zrow = jnp.zeros((ZROWS, FH), jnp.float32)
    zcnt = jnp.zeros((ZROWS,), jnp.float32)

    agg_a, agg_b, cnt_a, cnt_b = _sc_aggregate()(x2, srcg, dstg, zrow, zcnt)

    return _tc_call(
        x, agg_a, agg_b, cnt_a.reshape(N, 1), cnt_b.reshape(N, 1),
        W_l.T, W_r.T, W1.T, W2.T,
        b_l.reshape(1, F), b1.reshape(1, F), b2.reshape(1, F),
        ln1_g.reshape(1, F), ln1_b.reshape(1, F),
        ln2_g.reshape(1, F), ln2_b.reshape(1, F),
    )


# speedup vs baseline: 1.0787x; 1.0056x over previous
import functools

import jax
import jax.numpy as jnp
from jax import lax
from jax.experimental import pallas as pl
from jax.experimental.pallas import tpu as pltpu
from jax.experimental.pallas import tpu_sc as plsc

N = 10000
F = 256
FH = 128
E = 160000
NC = 2
NS = 16
CHUNK = 128
CH = 80
E_TILE = CH * CHUNK
E_PAD = NS * E_TILE
NPAD = 10112
ZROWS = NPAD // NS
OROWS = 624
TAIL = N - NS * OROWS

NBUF = 2
PH = CH // 2


def _sc_body(x2, srcg, dstg, zrow, zcnt, agg_a, agg_b, cnt_a, cnt_b,
             idx_v, rows_v, ones_v, stage_v, agg_s, cnt_s,
             gsem0, gsem1, ssem0, ssem1):
    gsems = (gsem0, gsem1)
    ssems = (ssem0, ssem1)
    c = lax.axis_index("c")
    s = lax.axis_index("s")

    def load_idx(p):
        pltpu.sync_copy(srcg.at[pl.ds((c * NS + s) * CH + p * PH, PH)],
                        idx_v.at[pl.ds(0, PH)])
        pltpu.sync_copy(dstg.at[pl.ds(s * CH + p * PH, PH)],
                        idx_v.at[pl.ds(PH, PH)])

    def gather(l, b):
        pltpu.async_copy(x2.at[idx_v.at[l]], rows_v.at[b], gsems[b])

    def pair(t, lookahead):
        for b in range(2):
            l = 2 * t + b
            pltpu.make_async_copy(x2.at[idx_v.at[l]], rows_v.at[b],
                                  gsems[b]).wait()
            pltpu.sync_copy(rows_v.at[b], agg_s.at[idx_v.at[PH + l]],
                            add=True)
            if lookahead:
                gather(2 * t + b + 2, b)
            @pl.when(c == b)
            def _():
                pltpu.sync_copy(ones_v, cnt_s.at[idx_v.at[PH + l]],
                                add=True)

    load_idx(0)
    gather(0, 0)
    gather(1, 1)
    pltpu.sync_copy(zrow, agg_s.at[pl.ds(s * ZROWS, ZROWS)])
    pltpu.sync_copy(zcnt, stage_v.at[pl.ds(0, ZROWS)])
    pltpu.sync_copy(stage_v.at[pl.ds(0, ZROWS)],
                    cnt_s.at[pl.ds(s * ZROWS, ZROWS)])
    for i in range(CHUNK // 16):
        ones_v[pl.ds(i * 16, 16)] = jnp.ones((16,), jnp.float32)
    plsc.subcore_barrier()

    for p in range(2):
        if p:
            load_idx(p)
            gather(0, 0)
            gather(1, 1)

        @pl.loop(0, PH // 2 - 1)
        def _ring(t):
            pair(t, lookahead=True)

        pair(PH // 2 - 1, lookahead=False)

    plsc.subcore_barrier()

    @pl.when(c == 0)
    def _():
        pltpu.sync_copy(agg_s.at[pl.ds(s * OROWS, OROWS)],
                        agg_a.at[pl.ds(s * OROWS, OROWS)])

        @pl.when(s == NS - 1)
        def _():
            pltpu.sync_copy(agg_s.at[pl.ds(NS * OROWS, TAIL)],
                            agg_a.at[pl.ds(NS * OROWS, TAIL)])

        @pl.when(s < 10)
        def _():
            pltpu.sync_copy(cnt_s.at[pl.ds(s * 1000, 1000)], stage_v)
            pltpu.sync_copy(stage_v, cnt_a.at[pl.ds(s * 1000, 1000)])

    @pl.when(c == 1)
    def _():
        pltpu.sync_copy(agg_s.at[pl.ds(s * OROWS, OROWS)],
                        agg_b.at[pl.ds(s * OROWS, OROWS)])

        @pl.when(s == NS - 1)
        def _():
            pltpu.sync_copy(agg_s.at[pl.ds(NS * OROWS, TAIL)],
                            agg_b.at[pl.ds(NS * OROWS, TAIL)])

        @pl.when(s < 10)
        def _():
            pltpu.sync_copy(cnt_s.at[pl.ds(s * 1000, 1000)], stage_v)
            pltpu.sync_copy(stage_v, cnt_b.at[pl.ds(s * 1000, 1000)])


@functools.lru_cache(maxsize=1)
def _sc_aggregate():
    mesh = plsc.VectorSubcoreMesh(core_axis_name="c", subcore_axis_name="s")
    return pl.kernel(
        _sc_body,
        out_type=(
            jax.ShapeDtypeStruct((N, FH), jnp.float32),
            jax.ShapeDtypeStruct((N, FH), jnp.float32),
            jax.ShapeDtypeStruct((N,), jnp.float32),
            jax.ShapeDtypeStruct((N,), jnp.float32),
        ),
        mesh=mesh,
        scratch_types=(
            pltpu.VMEM((2 * PH, CHUNK), jnp.int32),
            pltpu.VMEM((NBUF, CHUNK, FH), jnp.float32),
            pltpu.VMEM((CHUNK,), jnp.float32),
            pltpu.VMEM((1000,), jnp.float32),
            pltpu.VMEM_SHARED((NPAD, FH), jnp.float32),
            pltpu.VMEM_SHARED((NPAD,), jnp.float32),
            pltpu.SemaphoreType.DMA,
            pltpu.SemaphoreType.DMA,
            pltpu.SemaphoreType.DMA,
            pltpu.SemaphoreType.DMA,
        ),
    )


RB = 1000


def _tc_block(x_ref, aa_ref, ab_ref, ca_ref, cb_ref, wlt_ref, wrt_ref,
              w1t_ref, w2t_ref, bl_ref, b1_ref, b2_ref, g1_ref, bb1_ref,
              g2_ref, bb2_ref, out_ref):
    r = 1.0 / jnp.maximum(ca_ref[...] + cb_ref[...], 1.0)
    x = x_ref[...]
    y = jnp.dot(aa_ref[...] * r, wlt_ref[0:FH, :],
                preferred_element_type=jnp.float32)
    y = y + jnp.dot(ab_ref[...] * r, wlt_ref[FH:F, :],
                    preferred_element_type=jnp.float32)
    y = y + jnp.dot(x, wrt_ref[...], preferred_element_type=jnp.float32)
    z = y + bl_ref[...] + x
    mu = jnp.mean(z, axis=1, keepdims=True)
    zc = z - mu
    var = jnp.mean(zc * zc, axis=1, keepdims=True)
    y1 = zc * lax.rsqrt(var + 1e-5) * g1_ref[...] + bb1_ref[...]
    h = jnp.dot(y1, w1t_ref[...], preferred_element_type=jnp.float32)
    h = h + b1_ref[...]
    h = jnp.where(h > 0.0, h, jnp.exp(jnp.minimum(h, 0.0)) - 1.0)
    o = jnp.dot(h, w2t_ref[...], preferred_element_type=jnp.float32)
    o = o + b2_ref[...] + y1
    mu2 = jnp.mean(o, axis=1, keepdims=True)
    oc = o - mu2
    var2 = jnp.mean(oc * oc, axis=1, keepdims=True)
    out_ref[...] = oc * lax.rsqrt(var2 + 1e-5) * g2_ref[...] + bb2_ref[...]


def _row_spec(w):
    return pl.BlockSpec((RB, w), lambda i: (i, 0))


def _full_spec(h, w):
    return pl.BlockSpec((h, w), lambda i: (0, 0))


_tc_call = pl.pallas_call(
    _tc_block,
    grid=(N // RB,),
    in_specs=[
        _row_spec(F), _row_spec(FH), _row_spec(FH), _row_spec(1),
        _row_spec(1),
        _full_spec(F, F), _full_spec(F, F), _full_spec(F, F),
        _full_spec(F, F),
        _full_spec(1, F), _full_spec(1, F), _full_spec(1, F),
        _full_spec(1, F), _full_spec(1, F), _full_spec(1, F),
        _full_spec(1, F),
    ],
    out_specs=_row_spec(F),
    out_shape=jax.ShapeDtypeStruct((N, F), jnp.float32),
)


def kernel(x, edge_index, W_l, b_l, W_r, ln1_g, ln1_b, W1, b1, W2, b2,
           ln2_g, ln2_b):
    src = edge_index[0].astype(jnp.int32)
    dst = edge_index[1].astype(jnp.int32)
    pad = E_PAD - E
    src_p = jnp.concatenate([2 * src, jnp.zeros((pad,), jnp.int32)])
    dst_p = jnp.concatenate([dst, jnp.full((pad,), N, jnp.int32)])
    srcg = jnp.stack([src_p, src_p + 1]).reshape(NC * NS * CH, CHUNK)
    dstg = dst_p.reshape(NS * CH, CHUNK)
    x2 = x.reshape(2 * N, FH)
    zrow = jnp.zeros((ZROWS, FH), jnp.float32)
    zcnt = jnp.zeros((ZROWS,), jnp.float32)

    agg_a, agg_b, cnt_a, cnt_b = _sc_aggregate()(x2, srcg, dstg, zrow, zcnt)

    return _tc_call(
        x, agg_a, agg_b, cnt_a.reshape(N, 1), cnt_b.reshape(N, 1),
        W_l.T, W_r.T, W1.T, W2.T,
        b_l.reshape(1, F), b1.reshape(1, F), b2.reshape(1, F),
        ln1_g.reshape(1, F), ln1_b.reshape(1, F),
        ln2_g.reshape(1, F), ln2_b.reshape(1, F),
    )
